# baseline (speedup 1.0000x reference)
import jax
import jax.numpy as jnp
from jax.experimental import pallas as pl
from jax.experimental.pallas import tpu as pltpu
from functools import partial

BE_SUB = 1024
SUB = 4
BE = BE_SUB * SUB
W = 64

_TDIMS = (((0,), (0,)), ((), ()))


def _silu(v):
    return v * jax.nn.sigmoid(v)


def _fused_kernel(rbase_ref, rnw_ref, sbase_ref, snw_ref,
                  rrow_ref, srow_ref, drow_ref, x_ref,
                  A_ref, B_ref, C_ref, be1_ref, We2_ref, be2_ref,
                  Wi_ref, bi_ref, Wn1x_ref, Wn1a_ref, bn1_ref,
                  Wn2_ref, bn2_ref,
                  out_ref, agg_ref, xa_ref, xb_ref, accr_ref, accs_ref,
                  *, nblocks, n_nodes):
    i = pl.program_id(0)

    @pl.when(i == 0)
    def _():
        agg_ref[...] = jnp.zeros_like(agg_ref)
        xa_ref[...] = jnp.dot(x_ref[...], A_ref[...],
                              preferred_element_type=jnp.float32)
        xb_ref[...] = jnp.dot(x_ref[...], B_ref[...],
                              preferred_element_type=jnp.float32)

    row_iota = jax.lax.broadcasted_iota(jnp.int32, (W, BE_SUB), 0)

    def gather(src_ref, idx_row, base, nw, acc_ref, o):
        ohT0 = (row_iota == (idx_row - base)).astype(jnp.float32)
        acc_ref[o:o + BE_SUB, :] = jax.lax.dot_general(
            ohT0, src_ref[pl.ds(base, W), :], _TDIMS,
            preferred_element_type=jnp.float32)

        @pl.when(nw > 1)
        def _():
            def body(j, _):
                b = base + j * W
                ohT = (row_iota == (idx_row - b)).astype(jnp.float32)
                acc_ref[o:o + BE_SUB, :] += jax.lax.dot_general(
                    ohT, src_ref[pl.ds(b, W), :], _TDIMS,
                    preferred_element_type=jnp.float32)
                return 0
            jax.lax.fori_loop(1, nw, body, 0)
        return acc_ref[o:o + BE_SUB, :], ohT0

    for k in range(SUB):
        o = k * BE_SUB
        r_row = rrow_ref[k]
        s_row = srow_ref[k]
        d_row = drow_ref[k]
        rbase = rbase_ref[SUB * i + k] * 8
        rnw = rnw_ref[SUB * i + k]

        XrA, ohT0_r = gather(xa_ref, r_row, rbase, rnw, accr_ref, o)
        XsB, _ = gather(xb_ref, s_row, sbase_ref[SUB * i + k] * 8,
                        snw_ref[SUB * i + k], accs_ref, o)

        dC = jax.lax.dot_general(d_row, C_ref[...], _TDIMS,
                                 preferred_element_type=jnp.float32)
        m1 = _silu(XrA + XsB + dC + be1_ref[...])
        m2 = _silu(jnp.dot(m1, We2_ref[...],
                           preferred_element_type=jnp.float32)
                   + be2_ref[...])
        w = jax.nn.sigmoid(
            jnp.dot(m2, Wi_ref[...], preferred_element_type=jnp.float32)
            + bi_ref[...])
        m = m2 * w

        agg_ref[pl.ds(rbase, W), :] += jnp.dot(
            ohT0_r, m, preferred_element_type=jnp.float32)

        @pl.when(rnw > 1)
        def _():
            def scat(j, _):
                b = rbase + j * W
                ohT = (row_iota == (r_row - b)).astype(jnp.float32)
                agg_ref[pl.ds(b, W), :] += jnp.dot(
                    ohT, m, preferred_element_type=jnp.float32)
                return 0
            jax.lax.fori_loop(1, rnw, scat, 0)

    @pl.when(i == nblocks - 1)
    def _():
        xN = x_ref[:n_nodes, :]
        agg = agg_ref[:n_nodes, :]
        h = _silu(jnp.dot(xN, Wn1x_ref[...], preferred_element_type=jnp.float32)
                  + jnp.dot(agg, Wn1a_ref[...], preferred_element_type=jnp.float32)
                  + bn1_ref[...])
        out_ref[...] = (jnp.dot(h, Wn2_ref[...],
                                preferred_element_type=jnp.float32)
                        + bn2_ref[...])


@jax.jit
def kernel(x, distances, We1, be1, We2, be2, Wi, bi, Wn1, bn1, Wn2, bn2,
           edge_index):
    N, H = x.shape
    E = edge_index.shape[1]
    EP = pl.cdiv(E, BE) * BE
    nblocks = EP // BE
    nsub = EP // BE_SUB
    NP = ((N + W + 7) // 8) * 8

    receivers = edge_index[0].astype(jnp.int32)
    senders = edge_index[1].astype(jnp.int32)
    d1 = distances.astype(jnp.float32).reshape(E)
    if EP > E:
        receivers = jnp.concatenate(
            [receivers, jnp.full((EP - E,), N, jnp.int32)])
        senders = jnp.concatenate(
            [senders, jnp.full((EP - E,), N, jnp.int32)])
        d1 = jnp.concatenate([d1, jnp.zeros((EP - E,), jnp.float32)])

    xp = jnp.pad(x, ((0, NP - N), (0, 0)))

    rb = receivers.reshape(nsub, BE_SUB)
    sb = senders.reshape(nsub, BE_SUB)
    rbase = rb[:, 0] // 8
    rnw = (rb[:, -1] - rbase * 8) // W + 1
    sbase = sb[:, 0] // 8
    snw = (sb[:, -1] - sbase * 8) // W + 1

    r_rows = receivers.reshape(nsub, 1, BE_SUB)
    s_rows = senders.reshape(nsub, 1, BE_SUB)
    d_rows = d1.reshape(nsub, 1, BE_SUB)

    A = We1[:H]
    B = We1[H:2 * H]
    C = We1[2 * H:]
    Wn1x = Wn1[:H]
    Wn1a = Wn1[H:]

    grid_spec = pltpu.PrefetchScalarGridSpec(
        num_scalar_prefetch=4,
        grid=(nblocks,),
        in_specs=[
            pl.BlockSpec((SUB, 1, BE_SUB), lambda i, *_: (i, 0, 0)),
            pl.BlockSpec((SUB, 1, BE_SUB), lambda i, *_: (i, 0, 0)),
            pl.BlockSpec((SUB, 1, BE_SUB), lambda i, *_: (i, 0, 0)),
            pl.BlockSpec((NP, H), lambda i, *_: (0, 0)),
            pl.BlockSpec((H, H), lambda i, *_: (0, 0)),
            pl.BlockSpec((H, H), lambda i, *_: (0, 0)),
            pl.BlockSpec((1, H), lambda i, *_: (0, 0)),
            pl.BlockSpec((1, H), lambda i, *_: (0, 0)),
            pl.BlockSpec((H, H), lambda i, *_: (0, 0)),
            pl.BlockSpec((1, H), lambda i, *_: (0, 0)),
            pl.BlockSpec((H, 1), lambda i, *_: (0, 0)),
            pl.BlockSpec((1, 1), lambda i, *_: (0, 0)),
            pl.BlockSpec((H, H), lambda i, *_: (0, 0)),
            pl.BlockSpec((H, H), lambda i, *_: (0, 0)),
            pl.BlockSpec((1, H), lambda i, *_: (0, 0)),
            pl.BlockSpec((H, H), lambda i, *_: (0, 0)),
            pl.BlockSpec((1, H), lambda i, *_: (0, 0)),
        ],
        out_specs=pl.BlockSpec((N, H), lambda i, *_: (0, 0)),
        scratch_shapes=[pltpu.VMEM((NP, H), jnp.float32),
                        pltpu.VMEM((NP, H), jnp.float32),
                        pltpu.VMEM((NP, H), jnp.float32),
                        pltpu.VMEM((BE, H), jnp.float32),
                        pltpu.VMEM((BE, H), jnp.float32)],
    )

    return pl.pallas_call(
        partial(_fused_kernel, nblocks=nblocks, n_nodes=N),
        grid_spec=grid_spec,
        out_shape=jax.ShapeDtypeStruct((N, H), jnp.float32),
    )(rbase, rnw, sbase, snw,
      r_rows, s_rows, d_rows, xp,
      A, B, C, be1.reshape(1, H), We2, be2.reshape(1, H),
      Wi, bi.reshape(1, 1), Wn1x, Wn1a, bn1.reshape(1, H),
      Wn2, bn2.reshape(1, H))


# speedup vs baseline: 1.0057x; 1.0057x over previous
import jax
import jax.numpy as jnp
from jax.experimental import pallas as pl
from jax.experimental.pallas import tpu as pltpu
from functools import partial

BE_SUB = 1024
SUB = 2
BE = BE_SUB * SUB
W = 64

_TDIMS = (((0,), (0,)), ((), ()))


def _silu(v):
    return v * jax.nn.sigmoid(v)


def _fused_kernel(rbase_ref, rnw_ref, sbase_ref, snw_ref,
                  rrow_ref, srow_ref, drow_ref, x_ref,
                  A_ref, B_ref, C_ref, be1_ref, We2_ref, be2_ref,
                  Wi_ref, bi_ref, Wn1x_ref, Wn1a_ref, bn1_ref,
                  Wn2_ref, bn2_ref,
                  out_ref, agg_ref, xa_ref, xb_ref, accr_ref, accs_ref,
                  *, nblocks, n_nodes):
    i = pl.program_id(0)

    @pl.when(i == 0)
    def _():
        agg_ref[...] = jnp.zeros_like(agg_ref)
        xa_ref[...] = jnp.dot(x_ref[...], A_ref[...],
                              preferred_element_type=jnp.float32
                              ).astype(jnp.bfloat16)
        xb_ref[...] = jnp.dot(x_ref[...], B_ref[...],
                              preferred_element_type=jnp.float32
                              ).astype(jnp.bfloat16)

    row_iota = jax.lax.broadcasted_iota(jnp.int32, (W, BE_SUB), 0)

    def gather(src_ref, idx_row, base, nw, acc_ref, o):
        ohT0 = (row_iota == (idx_row - base)).astype(jnp.bfloat16)
        acc_ref[o:o + BE_SUB, :] = jax.lax.dot_general(
            ohT0, src_ref[pl.ds(base, W), :], _TDIMS,
            preferred_element_type=jnp.float32)

        @pl.when(nw > 1)
        def _():
            def body(j, _):
                b = base + j * W
                ohT = (row_iota == (idx_row - b)).astype(jnp.bfloat16)
                acc_ref[o:o + BE_SUB, :] += jax.lax.dot_general(
                    ohT, src_ref[pl.ds(b, W), :], _TDIMS,
                    preferred_element_type=jnp.float32)
                return 0
            jax.lax.fori_loop(1, nw, body, 0)
        return acc_ref[o:o + BE_SUB, :], ohT0

    for k in range(SUB):
        o = k * BE_SUB
        r_row = rrow_ref[k]
        s_row = srow_ref[k]
        d_row = drow_ref[k]
        rbase = rbase_ref[SUB * i + k] * 16
        rnw = rnw_ref[SUB * i + k]

        XrA, ohT0_r = gather(xa_ref, r_row, rbase, rnw, accr_ref, o)
        XsB, _ = gather(xb_ref, s_row, sbase_ref[SUB * i + k] * 16,
                        snw_ref[SUB * i + k], accs_ref, o)

        dC = jax.lax.dot_general(d_row, C_ref[...], _TDIMS,
                                 preferred_element_type=jnp.float32)
        m1 = _silu(XrA + XsB + dC + be1_ref[...])
        m2 = _silu(jnp.dot(m1, We2_ref[...],
                           preferred_element_type=jnp.float32)
                   + be2_ref[...])
        w = jax.nn.sigmoid(
            jnp.dot(m2, Wi_ref[...], preferred_element_type=jnp.float32)
            + bi_ref[...])
        m = (m2 * w).astype(jnp.bfloat16)

        agg_ref[pl.ds(rbase, W), :] += jnp.dot(
            ohT0_r, m, preferred_element_type=jnp.float32)

        @pl.when(rnw > 1)
        def _():
            def scat(j, _):
                b = rbase + j * W
                ohT = (row_iota == (r_row - b)).astype(jnp.bfloat16)
                agg_ref[pl.ds(b, W), :] += jnp.dot(
                    ohT, m, preferred_element_type=jnp.float32)
                return 0
            jax.lax.fori_loop(1, rnw, scat, 0)

    @pl.when(i == nblocks - 1)
    def _():
        xN = x_ref[:n_nodes, :]
        agg = agg_ref[:n_nodes, :]
        h = _silu(jnp.dot(xN, Wn1x_ref[...], preferred_element_type=jnp.float32)
                  + jnp.dot(agg, Wn1a_ref[...], preferred_element_type=jnp.float32)
                  + bn1_ref[...])
        out_ref[...] = (jnp.dot(h, Wn2_ref[...],
                                preferred_element_type=jnp.float32)
                        + bn2_ref[...])


@jax.jit
def kernel(x, distances, We1, be1, We2, be2, Wi, bi, Wn1, bn1, Wn2, bn2,
           edge_index):
    N, H = x.shape
    E = edge_index.shape[1]
    EP = pl.cdiv(E, BE) * BE
    nblocks = EP // BE
    nsub = EP // BE_SUB
    NP = ((N + W + 7) // 8) * 8

    receivers = edge_index[0].astype(jnp.int32)
    senders = edge_index[1].astype(jnp.int32)
    d1 = distances.astype(jnp.float32).reshape(E)
    if EP > E:
        receivers = jnp.concatenate(
            [receivers, jnp.full((EP - E,), N, jnp.int32)])
        senders = jnp.concatenate(
            [senders, jnp.full((EP - E,), N, jnp.int32)])
        d1 = jnp.concatenate([d1, jnp.zeros((EP - E,), jnp.float32)])

    xp = jnp.pad(x, ((0, NP - N), (0, 0)))

    rb = receivers.reshape(nsub, BE_SUB)
    sb = senders.reshape(nsub, BE_SUB)
    rbase = rb[:, 0] // 16
    rnw = (rb[:, -1] - rbase * 16) // W + 1
    sbase = sb[:, 0] // 16
    snw = (sb[:, -1] - sbase * 16) // W + 1

    r_rows = receivers.reshape(nsub, 1, BE_SUB)
    s_rows = senders.reshape(nsub, 1, BE_SUB)
    d_rows = d1.reshape(nsub, 1, BE_SUB)

    A = We1[:H]
    B = We1[H:2 * H]
    C = We1[2 * H:]
    Wn1x = Wn1[:H]
    Wn1a = Wn1[H:]

    grid_spec = pltpu.PrefetchScalarGridSpec(
        num_scalar_prefetch=4,
        grid=(nblocks,),
        in_specs=[
            pl.BlockSpec((SUB, 1, BE_SUB), lambda i, *_: (i, 0, 0)),
            pl.BlockSpec((SUB, 1, BE_SUB), lambda i, *_: (i, 0, 0)),
            pl.BlockSpec((SUB, 1, BE_SUB), lambda i, *_: (i, 0, 0)),
            pl.BlockSpec((NP, H), lambda i, *_: (0, 0)),
            pl.BlockSpec((H, H), lambda i, *_: (0, 0)),
            pl.BlockSpec((H, H), lambda i, *_: (0, 0)),
            pl.BlockSpec((1, H), lambda i, *_: (0, 0)),
            pl.BlockSpec((1, H), lambda i, *_: (0, 0)),
            pl.BlockSpec((H, H), lambda i, *_: (0, 0)),
            pl.BlockSpec((1, H), lambda i, *_: (0, 0)),
            pl.BlockSpec((H, 1), lambda i, *_: (0, 0)),
            pl.BlockSpec((1, 1), lambda i, *_: (0, 0)),
            pl.BlockSpec((H, H), lambda i, *_: (0, 0)),
            pl.BlockSpec((H, H), lambda i, *_: (0, 0)),
            pl.BlockSpec((1, H), lambda i, *_: (0, 0)),
            pl.BlockSpec((H, H), lambda i, *_: (0, 0)),
            pl.BlockSpec((1, H), lambda i, *_: (0, 0)),
        ],
        out_specs=pl.BlockSpec((N, H), lambda i, *_: (0, 0)),
        scratch_shapes=[pltpu.VMEM((NP, H), jnp.float32),
                        pltpu.VMEM((NP, H), jnp.bfloat16),
                        pltpu.VMEM((NP, H), jnp.bfloat16),
                        pltpu.VMEM((BE, H), jnp.float32),
                        pltpu.VMEM((BE, H), jnp.float32)],
    )

    return pl.pallas_call(
        partial(_fused_kernel, nblocks=nblocks, n_nodes=N),
        grid_spec=grid_spec,
        out_shape=jax.ShapeDtypeStruct((N, H), jnp.float32),
    )(rbase, rnw, sbase, snw,
      r_rows, s_rows, d_rows, xp,
      A, B, C, be1.reshape(1, H), We2, be2.reshape(1, H),
      Wi, bi.reshape(1, 1), Wn1x, Wn1a, bn1.reshape(1, H),
      Wn2, bn2.reshape(1, H))
